# baseline (speedup 1.0000x reference)
import functools

import jax
import jax.numpy as jnp
from jax import lax
from jax.experimental import pallas as pl
from jax.experimental.pallas import tpu as pltpu
from jax.experimental.pallas import tpu_sc as plsc

VOCAB_N = 1_000_000
D = 64
BATCH_N = 16384
HIST_N = 200
B_TOTAL = BATCH_N * HIST_N

NW = 32
PER_W = B_TOTAL // NW
SUB = 128
NSUB = PER_W // SUB
LANES = 16
NCREG = D // LANES
NRING = 3

_MEAN_BLK = 8000
_MEAN_GRID = VOCAB_N // _MEAN_BLK


def _mean_body(t_ref, c_ref):
    i = pl.program_id(0)

    @pl.when(i == 0)
    def _():
        c_ref[...] = jnp.zeros_like(c_ref)

    s = jnp.sum(t_ref[...], axis=0, keepdims=True)
    c_ref[...] += jnp.broadcast_to(jnp.concatenate([s, s], axis=1), (8, 2 * D))

    @pl.when(i == _MEAN_GRID - 1)
    def _():
        c_ref[...] = c_ref[...] * (1.0 / VOCAB_N)


def _tc_mean(table):
    return pl.pallas_call(
        _mean_body,
        grid=(_MEAN_GRID,),
        in_specs=[pl.BlockSpec((_MEAN_BLK, D), lambda i: (i, 0))],
        out_specs=pl.BlockSpec((8, 2 * D), lambda i: (0, 0)),
        out_shape=jax.ShapeDtypeStruct((8, 2 * D), jnp.float32),
    )(table)


_mesh = plsc.VectorSubcoreMesh(core_axis_name="c", subcore_axis_name="s")


@functools.partial(
    pl.kernel,
    mesh=_mesh,
    compiler_params=pltpu.CompilerParams(
        use_tc_tiling_on_sc=False, needs_layout_passes=False),
    out_type=jax.ShapeDtypeStruct((B_TOTAL, 2 * D), jnp.float32),
    scratch_types=[
        pltpu.VMEM((NRING, 1, SUB), jnp.int32),
        pltpu.VMEM((NRING, SUB, 2 * D), jnp.float32),
        pltpu.VMEM((NRING, SUB, D), jnp.float32),
        pltpu.VMEM((8, 2 * D), jnp.float32),
        pltpu.SemaphoreType.DMA,
        pltpu.SemaphoreType.DMA,
        pltpu.SemaphoreType.DMA,
    ],
)
def _sc_gather_sub(x_hbm, table_hbm, center_hbm, out_hbm,
                   idx_v, rows_v, stage_v, center_v,
                   sem_i, sem_g, sem_s):
    wid = lax.axis_index("s") * 2 + lax.axis_index("c")
    xbase = wid * NSUB
    obase = wid * PER_W

    pltpu.sync_copy(center_hbm, center_v)
    cregs = [center_v[0, pl.ds(LANES * c, LANES)] for c in range(NCREG)]

    def fire_idx(slot, s):
        pltpu.async_copy(x_hbm.at[pl.ds(xbase + s, 1)], idx_v.at[slot], sem_i)

    def wait_idx(slot):
        pltpu.make_async_copy(
            x_hbm.at[pl.ds(0, 1)], idx_v.at[slot], sem_i).wait()

    def fire_gather(slot):
        pltpu.async_copy(
            table_hbm.at[idx_v.at[slot, 0]], rows_v.at[slot], sem_g)

    def wait_gather(slot):
        pltpu.make_async_copy(
            table_hbm.at[idx_v.at[slot, 0]], rows_v.at[slot], sem_g).wait()

    def fire_scatter(slot, s):
        pltpu.async_copy(
            stage_v.at[slot],
            out_hbm.at[pl.ds(obase + s * SUB, SUB), pl.ds(0, D)],
            sem_s)

    def wait_scatter(slot):
        pltpu.make_async_copy(
            stage_v.at[slot],
            out_hbm.at[pl.ds(0, SUB), pl.ds(0, D)],
            sem_s).wait()

    def process(slot):
        def row(j, carry):
            for c in range(NCREG):
                sl = pl.ds(LANES * c, LANES)
                stage_v[slot, j, sl] = rows_v[slot, j, sl] - cregs[c]
            return carry

        lax.fori_loop(0, SUB, row, 0, unroll=4)

    for v in range(2):
        fire_idx(v, v)
        wait_idx(v)
        fire_gather(v)
    fire_idx(2, 2)

    def body(u, carry):
        slot = lax.rem(u, NRING)

        @pl.when(u + 2 < NSUB)
        def _():
            s2 = lax.rem(u + 2, NRING)
            wait_idx(s2)
            fire_gather(s2)

        @pl.when(u + 3 < NSUB)
        def _():
            fire_idx(slot, u + 3)

        wait_gather(slot)

        @pl.when(u >= 2)
        def _():
            wait_scatter(lax.rem(u + 1, NRING))

        process(slot)
        fire_scatter(slot, u)
        return carry

    lax.fori_loop(0, NSUB, body, 0)
    wait_scatter(0)
    wait_scatter(1)


def kernel(x, table):
    center = _tc_mean(table)
    x2 = x.reshape(-1).astype(jnp.int32).reshape(B_TOTAL // SUB, SUB)
    table_pad = jnp.pad(table, ((0, 0), (0, D)))
    out2d = _sc_gather_sub(x2, table_pad, center)
    return out2d.reshape(BATCH_N, HIST_N, 2 * D)[:, :, :D]


# speedup vs baseline: 1.5161x; 1.5161x over previous
import functools

import jax
import jax.numpy as jnp
from jax import lax
from jax.experimental import pallas as pl
from jax.experimental.pallas import tpu as pltpu
from jax.experimental.pallas import tpu_sc as plsc

VOCAB_N = 1_000_000
D = 64
BATCH_N = 16384
HIST_N = 200
B_TOTAL = BATCH_N * HIST_N

NW = 32
PER_W = B_TOTAL // NW
SUB = 128
NSUB = PER_W // SUB
LANES = 16
NCREG = D // LANES
NRING = 6

_MEAN_BLK = 8000
_MEAN_GRID = VOCAB_N // _MEAN_BLK


def _mean_body(t_ref, c_ref):
    i = pl.program_id(0)

    @pl.when(i == 0)
    def _():
        c_ref[...] = jnp.zeros_like(c_ref)

    s = jnp.sum(t_ref[...], axis=0, keepdims=True)
    c_ref[...] += jnp.broadcast_to(jnp.concatenate([s, s], axis=1), (8, 2 * D))

    @pl.when(i == _MEAN_GRID - 1)
    def _():
        c_ref[...] = c_ref[...] * (1.0 / VOCAB_N)


def _tc_mean(table):
    return pl.pallas_call(
        _mean_body,
        grid=(_MEAN_GRID,),
        in_specs=[pl.BlockSpec((_MEAN_BLK, D), lambda i: (i, 0))],
        out_specs=pl.BlockSpec((8, 2 * D), lambda i: (0, 0)),
        out_shape=jax.ShapeDtypeStruct((8, 2 * D), jnp.float32),
    )(table)


_mesh = plsc.VectorSubcoreMesh(core_axis_name="c", subcore_axis_name="s")


@functools.partial(
    pl.kernel,
    mesh=_mesh,
    compiler_params=pltpu.CompilerParams(
        use_tc_tiling_on_sc=False, needs_layout_passes=False),
    out_type=jax.ShapeDtypeStruct((B_TOTAL, 2 * D), jnp.float32),
    scratch_types=[
        pltpu.VMEM((NRING, 1, SUB), jnp.int32),
        pltpu.VMEM((NRING, SUB, 2 * D), jnp.float32),
        pltpu.VMEM((8, 2 * D), jnp.float32),
        pltpu.SemaphoreType.DMA,
        pltpu.SemaphoreType.DMA,
        pltpu.SemaphoreType.DMA,
    ],
)
def _sc_gather_sub(x_hbm, table_hbm, center_hbm, out_hbm,
                   idx_v, rows_v, center_v,
                   sem_i, sem_g, sem_s):
    wid = lax.axis_index("s") * 2 + lax.axis_index("c")
    xbase = wid * NSUB
    obase = wid * PER_W

    pltpu.sync_copy(center_hbm, center_v)
    cregs = [center_v[0, pl.ds(LANES * c, LANES)] for c in range(NCREG)]

    def fire_idx(slot, s):
        pltpu.async_copy(x_hbm.at[pl.ds(xbase + s, 1)], idx_v.at[slot], sem_i)

    def wait_idx(slot):
        pltpu.make_async_copy(
            x_hbm.at[pl.ds(0, 1)], idx_v.at[slot], sem_i).wait()

    def fire_gather(slot):
        pltpu.async_copy(
            table_hbm.at[idx_v.at[slot, 0]], rows_v.at[slot], sem_g)

    def wait_gather(slot):
        pltpu.make_async_copy(
            table_hbm.at[idx_v.at[slot, 0]], rows_v.at[slot], sem_g).wait()

    def fire_scatter(slot, s):
        pltpu.async_copy(
            rows_v.at[slot, :, pl.ds(0, D)],
            out_hbm.at[pl.ds(obase + s * SUB, SUB), pl.ds(0, D)],
            sem_s)

    def wait_scatter(slot):
        pltpu.make_async_copy(
            rows_v.at[slot, :, pl.ds(0, D)],
            out_hbm.at[pl.ds(0, SUB), pl.ds(0, D)],
            sem_s).wait()

    def process(slot):
        def row(j, carry):
            for c in range(NCREG):
                sl = pl.ds(LANES * c, LANES)
                rows_v[slot, j, sl] = rows_v[slot, j, sl] - cregs[c]
            return carry

        lax.fori_loop(0, SUB, row, 0, unroll=4)

    for v in range(4):
        fire_idx(v, v)
        wait_idx(v)
        fire_gather(v)
    fire_idx(4, 4)
    fire_idx(5, 5)

    def body(u, carry):
        slot = lax.rem(u, NRING)

        @pl.when(u >= 2)
        def _():
            wait_scatter(lax.rem(u + 4, NRING))

        @pl.when(u + 4 < NSUB)
        def _():
            s4 = lax.rem(u + 4, NRING)
            wait_idx(s4)
            fire_gather(s4)

        wait_gather(slot)

        @pl.when(u + 6 < NSUB)
        def _():
            fire_idx(slot, u + 6)

        process(slot)
        fire_scatter(slot, u)
        return carry

    lax.fori_loop(0, NSUB, body, 0)
    wait_scatter(lax.rem(NSUB - 2, NRING))
    wait_scatter(lax.rem(NSUB - 1, NRING))


def kernel(x, table):
    center = _tc_mean(table)
    x2 = x.reshape(-1).astype(jnp.int32).reshape(B_TOTAL // SUB, SUB)
    table_pad = jnp.pad(table, ((0, 0), (0, D)))
    out2d = _sc_gather_sub(x2, table_pad, center)
    return out2d.reshape(BATCH_N, HIST_N, 2 * D)[:, :, :D]


# speedup vs baseline: 1.6408x; 1.0822x over previous
import functools

import jax
import jax.numpy as jnp
from jax import lax
from jax.experimental import pallas as pl
from jax.experimental.pallas import tpu as pltpu
from jax.experimental.pallas import tpu_sc as plsc

VOCAB_N = 1_000_000
D = 64
BATCH_N = 16384
HIST_N = 200
B_TOTAL = BATCH_N * HIST_N

NW = 32
PER_W = B_TOTAL // NW
SUB = 128
NSUB = PER_W // SUB
LANES = 16
NCREG = D // LANES
NRING = 6

_MEAN_BLK = 8000
_MEAN_GRID = VOCAB_N // _MEAN_BLK


def _mean_body(t_ref, pad_ref, c_ref):
    i = pl.program_id(0)
    t = t_ref[...]
    pad_ref[...] = jnp.concatenate([t, jnp.zeros_like(t)], axis=1)

    @pl.when(i == 0)
    def _():
        c_ref[...] = jnp.zeros_like(c_ref)

    s = jnp.sum(t, axis=0, keepdims=True)
    c_ref[...] += jnp.broadcast_to(jnp.concatenate([s, s], axis=1), (8, 2 * D))

    @pl.when(i == _MEAN_GRID - 1)
    def _():
        c_ref[...] = c_ref[...] * (1.0 / VOCAB_N)


def _tc_mean(table):
    return pl.pallas_call(
        _mean_body,
        grid=(_MEAN_GRID,),
        in_specs=[pl.BlockSpec((_MEAN_BLK, D), lambda i: (i, 0))],
        out_specs=[
            pl.BlockSpec((_MEAN_BLK, 2 * D), lambda i: (i, 0)),
            pl.BlockSpec((8, 2 * D), lambda i: (0, 0)),
        ],
        out_shape=[
            jax.ShapeDtypeStruct((VOCAB_N, 2 * D), jnp.float32),
            jax.ShapeDtypeStruct((8, 2 * D), jnp.float32),
        ],
    )(table)


_mesh = plsc.VectorSubcoreMesh(core_axis_name="c", subcore_axis_name="s")


@functools.partial(
    pl.kernel,
    mesh=_mesh,
    compiler_params=pltpu.CompilerParams(
        use_tc_tiling_on_sc=False, needs_layout_passes=False),
    out_type=jax.ShapeDtypeStruct((B_TOTAL, 2 * D), jnp.float32),
    scratch_types=[
        pltpu.VMEM((NRING, 1, SUB), jnp.int32),
        pltpu.VMEM((NRING, SUB, 2 * D), jnp.float32),
        pltpu.VMEM((8, 2 * D), jnp.float32),
        pltpu.SemaphoreType.DMA,
        pltpu.SemaphoreType.DMA,
        pltpu.SemaphoreType.DMA,
    ],
)
def _sc_gather_sub(x_hbm, table_hbm, center_hbm, out_hbm,
                   idx_v, rows_v, center_v,
                   sem_i, sem_g, sem_s):
    wid = lax.axis_index("s") * 2 + lax.axis_index("c")
    xbase = wid * NSUB
    obase = wid * PER_W

    pltpu.sync_copy(center_hbm, center_v)
    cregs = [center_v[0, pl.ds(LANES * c, LANES)] for c in range(NCREG)]

    def fire_idx(slot, s):
        pltpu.async_copy(x_hbm.at[pl.ds(xbase + s, 1)], idx_v.at[slot], sem_i)

    def wait_idx(slot):
        pltpu.make_async_copy(
            x_hbm.at[pl.ds(0, 1)], idx_v.at[slot], sem_i).wait()

    def fire_gather(slot):
        pltpu.async_copy(
            table_hbm.at[idx_v.at[slot, 0]], rows_v.at[slot], sem_g)

    def wait_gather(slot):
        pltpu.make_async_copy(
            table_hbm.at[idx_v.at[slot, 0]], rows_v.at[slot], sem_g).wait()

    def fire_scatter(slot, s):
        pltpu.async_copy(
            rows_v.at[slot, :, pl.ds(0, D)],
            out_hbm.at[pl.ds(obase + s * SUB, SUB), pl.ds(0, D)],
            sem_s)

    def wait_scatter(slot):
        pltpu.make_async_copy(
            rows_v.at[slot, :, pl.ds(0, D)],
            out_hbm.at[pl.ds(0, SUB), pl.ds(0, D)],
            sem_s).wait()

    def process(slot):
        def row(j, carry):
            for c in range(NCREG):
                sl = pl.ds(LANES * c, LANES)
                rows_v[slot, j, sl] = rows_v[slot, j, sl] - cregs[c]
            return carry

        lax.fori_loop(0, SUB, row, 0, unroll=4)

    for v in range(4):
        fire_idx(v, v)
        wait_idx(v)
        fire_gather(v)
    fire_idx(4, 4)
    fire_idx(5, 5)

    def body(u, carry):
        slot = lax.rem(u, NRING)

        @pl.when(u >= 2)
        def _():
            wait_scatter(lax.rem(u + 4, NRING))

        @pl.when(u + 4 < NSUB)
        def _():
            s4 = lax.rem(u + 4, NRING)
            wait_idx(s4)
            fire_gather(s4)

        wait_gather(slot)

        @pl.when(u + 6 < NSUB)
        def _():
            fire_idx(slot, u + 6)

        process(slot)
        fire_scatter(slot, u)
        return carry

    lax.fori_loop(0, NSUB, body, 0)
    wait_scatter(lax.rem(NSUB - 2, NRING))
    wait_scatter(lax.rem(NSUB - 1, NRING))


def kernel(x, table):
    table_pad, center = _tc_mean(table)
    x2 = x.reshape(-1).astype(jnp.int32).reshape(B_TOTAL // SUB, SUB)
    out2d = _sc_gather_sub(x2, table_pad, center)
    return out2d.reshape(BATCH_N, HIST_N, 2 * D)[:, :, :D]
